# baseline (speedup 1.0000x reference)
import functools

import jax
import jax.numpy as jnp
import numpy as np
from jax import lax
from jax.experimental import pallas as pl
from jax.experimental.pallas import tpu as pltpu
from jax.experimental.pallas import tpu_sc as plsc

NUM_BINS = 30
EPS_CLIP = 1e-7
_F16_EPS = float(np.finfo(np.float16).eps)
_SCALE = NUM_BINS - _F16_EPS
_PAD = 32

_NC, _NS, _L = 2, 16, 16
_NW = _NC * _NS
_CH = 16384
_REG = 33
_TBL = 544
_UNR = 8
_HSZ = _TBL * _UNR


def _sc_hist_body(p_hbm, t_hbm, out_hbm, pbuf, tbuf, hist, stage,
                  sp0, sp1, st0, st1):
    wid = lax.axis_index("c") * _NS + lax.axis_index("s")
    n = p_hbm.shape[0]
    per_w = n // _NW
    nchunk = per_w // _CH
    base = wid * per_w

    zero16 = jnp.zeros((_L,), jnp.float32)
    for k in range(_HSZ // _L):
        hist[pl.ds(k * _L, _L)] = zero16

    lane_offs = [
        lax.iota(jnp.int32, _L) * _REG + u * _TBL for u in range(_UNR)
    ]
    ones = jnp.ones((_L,), jnp.float32)

    def issue(c, slot, semp, semt):
        off = base + c * _CH
        pltpu.async_copy(p_hbm.at[pl.ds(off, _CH)], pbuf.at[slot], semp)
        pltpu.async_copy(t_hbm.at[pl.ds(off, _CH)], tbuf.at[slot], semt)

    def wait(slot, semp, semt):
        pltpu.make_async_copy(
            p_hbm.at[pl.ds(0, _CH)], pbuf.at[slot], semp).wait()
        pltpu.make_async_copy(
            t_hbm.at[pl.ds(0, _CH)], tbuf.at[slot], semt).wait()

    def compute(slot):
        def body(g, _):
            b0 = g * (_UNR * _L)
            for u in range(_UNR):
                pv = pbuf[slot, pl.ds(b0 + u * _L, _L)]
                tv = tbuf[slot, pl.ds(b0 + u * _L, _L)]
                idx = (jnp.abs(pv - tv) * _SCALE).astype(jnp.int32)
                plsc.addupdate_scatter(hist, [idx + lane_offs[u]], ones)
            return 0

        lax.fori_loop(0, _CH // (_UNR * _L), body, 0)

    issue(0, 0, sp0, st0)

    def chunk_pair(gp, _):
        c0 = gp * 2
        issue(c0 + 1, 1, sp1, st1)
        wait(0, sp0, st0)
        compute(0)
        issue(jnp.minimum(c0 + 2, nchunk - 1), 0, sp0, st0)
        wait(1, sp1, st1)
        compute(1)
        return 0

    lax.fori_loop(0, nchunk // 2, chunk_pair, 0)
    wait(0, sp0, st0)

    acc_a = zero16
    acc_b = zero16
    for u in range(_UNR):
        for l in range(_L):
            acc_a = acc_a + hist[pl.ds(u * _TBL + l * _REG, _L)]
            acc_b = acc_b + hist[pl.ds(u * _TBL + l * _REG + _L, _L)]
    stage[pl.ds(0, _L)] = acc_a
    stage[pl.ds(_L, _L)] = acc_b
    pltpu.sync_copy(stage, out_hbm.at[wid])


_sc_hist = functools.partial(
    pl.kernel,
    out_type=jax.ShapeDtypeStruct((_NW, _PAD), jnp.float32),
    mesh=plsc.VectorSubcoreMesh(core_axis_name="c", subcore_axis_name="s"),
    compiler_params=pltpu.CompilerParams(
        use_tc_tiling_on_sc=False, needs_layout_passes=False
    ),
    scratch_types=[
        pltpu.VMEM((2, _CH), jnp.float32),
        pltpu.VMEM((2, _CH), jnp.float32),
        pltpu.VMEM((_HSZ,), jnp.float32),
        pltpu.VMEM((_PAD,), jnp.float32),
        pltpu.SemaphoreType.DMA,
        pltpu.SemaphoreType.DMA,
        pltpu.SemaphoreType.DMA,
        pltpu.SemaphoreType.DMA,
    ],
)(_sc_hist_body)


def _loss_body(num_calc, hist_ref, p_ref, t_ref, out_ref):
    counts = jnp.sum(hist_ref[...], axis=0, keepdims=True)
    nvalid = jnp.sum((counts > 0).astype(jnp.float32))
    scale = num_calc * nvalid
    recip = jnp.where(counts > 0, scale / jnp.maximum(counts, 1.0), 0.0)

    p = p_ref[...]
    t = t_ref[...]
    bm, bn = p.shape
    x = jnp.abs(p - t) * _SCALE
    idx = x.astype(jnp.int32)
    table = jnp.concatenate(
        [recip, jnp.zeros((1, 128 - _PAD), jnp.float32)], axis=1
    )
    table = jnp.broadcast_to(table, (bm, 128))
    cols = []
    for k in range(bn // 128):
        idxk = idx[:, k * 128 : (k + 1) * 128]
        cols.append(
            jnp.take_along_axis(table, idxk, axis=1, mode="promise_in_bounds")
        )
    w = jnp.concatenate(cols, axis=1)

    pc = jnp.clip(p, EPS_CLIP, 1.0 - EPS_CLIP)
    bce = -(t * jnp.log(pc) + (1.0 - t) * jnp.log(1.0 - pc))
    out_ref[...] = bce * w


def kernel(pconf, gconf):
    m, n = pconf.shape
    hist = _sc_hist(pconf.reshape(-1), gconf.reshape(-1))

    bm = 256
    grid = (m // bm,)
    blk = pl.BlockSpec((bm, n), lambda i: (i, 0))
    num_calc = np.float32(m * n)
    loss = pl.pallas_call(
        functools.partial(_loss_body, num_calc),
        grid=grid,
        in_specs=[pl.BlockSpec((_NW, _PAD), lambda i: (0, 0)), blk, blk],
        out_specs=blk,
        out_shape=jax.ShapeDtypeStruct((m, n), jnp.float32),
    )(hist, pconf, gconf)
    return loss


# speedup vs baseline: 4.7677x; 4.7677x over previous
import functools

import jax
import jax.numpy as jnp
import numpy as np
from jax.experimental import pallas as pl

NUM_BINS = 30
EPS_CLIP = 1e-7
_F16_EPS = float(np.finfo(np.float16).eps)
_SCALE = NUM_BINS - _F16_EPS
_PAD = 32


def _csa_push(acc, w, word):
    lst = acc.setdefault(w, [])
    lst.append(word)
    while len(lst) >= 3:
        a = lst.pop()
        b = lst.pop()
        c = lst.pop()
        t = a ^ b
        lst.append(t ^ c)
        _csa_push(acc, w + 1, (a & b) | (t & c))


def _hist_body(p_ref, t_ref, hist_ref):
    i = pl.program_id(0)
    bm, bn = p_ref.shape
    acc = {}
    for r in range(0, bm, 8):
        p = p_ref[pl.ds(r, 8), :]
        t = t_ref[pl.ds(r, 8), :]
        ind = (jnp.abs(p - t) * _SCALE).astype(jnp.int32)
        pw = (jnp.int32(2) << ind) - 1
        for k in range(0, bn, 128):
            _csa_push(acc, 0, pw[:, k : k + 128])

    planes = [(w, word) for w, lst in acc.items() for word in lst]
    ge = []
    for b in range(NUM_BINS):
        tot = None
        for w, word in planes:
            term = ((word >> b) & 1) << w
            tot = term if tot is None else tot + term
        ge.append(jnp.sum(tot))
    ge.append(jnp.int32(0))
    counts = [(ge[b] - ge[b + 1]).astype(jnp.float32) for b in range(NUM_BINS)]
    counts += [jnp.float32(0.0)] * (_PAD - NUM_BINS)
    h = jnp.stack(counts).reshape(1, _PAD)

    @pl.when(i == 0)
    def _():
        hist_ref[...] = h

    @pl.when(i > 0)
    def _():
        hist_ref[...] += h


def _loss_body(num_calc, hist_ref, p_ref, t_ref, out_ref):
    counts = hist_ref[...]
    nvalid = jnp.sum((counts > 0).astype(jnp.float32))
    scale = num_calc * nvalid
    recip = jnp.where(counts > 0, scale / jnp.maximum(counts, 1.0), 0.0)

    p = p_ref[...]
    t = t_ref[...]
    bm, bn = p.shape
    x = jnp.abs(p - t) * _SCALE
    idx = x.astype(jnp.int32)
    table = jnp.concatenate(
        [recip, jnp.zeros((1, 128 - _PAD), jnp.float32)], axis=1
    )
    table = jnp.broadcast_to(table, (bm, 128))
    cols = []
    for k in range(bn // 128):
        idxk = idx[:, k * 128 : (k + 1) * 128]
        cols.append(
            jnp.take_along_axis(table, idxk, axis=1, mode="promise_in_bounds")
        )
    w = jnp.concatenate(cols, axis=1)

    pc = jnp.clip(p, EPS_CLIP, 1.0 - EPS_CLIP)
    bce = -(t * jnp.log(pc) + (1.0 - t) * jnp.log(1.0 - pc))
    out_ref[...] = bce * w


def kernel(pconf, gconf):
    m, n = pconf.shape

    bm1 = 128
    blk1 = pl.BlockSpec((bm1, n), lambda i: (i, 0))
    hist = pl.pallas_call(
        _hist_body,
        grid=(m // bm1,),
        in_specs=[blk1, blk1],
        out_specs=pl.BlockSpec((1, _PAD), lambda i: (0, 0)),
        out_shape=jax.ShapeDtypeStruct((1, _PAD), jnp.float32),
    )(pconf, gconf)

    bm2 = 256
    blk2 = pl.BlockSpec((bm2, n), lambda i: (i, 0))
    num_calc = np.float32(m * n)
    loss = pl.pallas_call(
        functools.partial(_loss_body, num_calc),
        grid=(m // bm2,),
        in_specs=[pl.BlockSpec((1, _PAD), lambda i: (0, 0)), blk2, blk2],
        out_specs=blk2,
        out_shape=jax.ShapeDtypeStruct((m, n), jnp.float32),
    )(hist, pconf, gconf)
    return loss


# speedup vs baseline: 5.1141x; 1.0726x over previous
import functools

import jax
import jax.numpy as jnp
import numpy as np
from jax.experimental import pallas as pl

NUM_BINS = 30
EPS_CLIP = 1e-7
_F16_EPS = float(np.finfo(np.float16).eps)
_SCALE = NUM_BINS - _F16_EPS
_PAD = 32


def _csa_push(acc, w, word):
    lst = acc.setdefault(w, [])
    lst.append(word)
    while len(lst) >= 3:
        a = lst.pop()
        b = lst.pop()
        c = lst.pop()
        t = a ^ b
        lst.append(t ^ c)
        _csa_push(acc, w + 1, (a & b) | (t & c))


def _hist_body(p_ref, t_ref, hist_ref):
    i = pl.program_id(0)
    bm, bn = p_ref.shape
    acc = {}
    for r in range(0, bm, 8):
        p = p_ref[pl.ds(r, 8), :]
        t = t_ref[pl.ds(r, 8), :]
        ind = (jnp.abs(p - t) * _SCALE).astype(jnp.int32)
        pw = (jnp.int32(2) << ind) - 1
        for k in range(0, bn, 128):
            _csa_push(acc, 0, pw[:, k : k + 128])

    planes = [(w, word) for w, lst in acc.items() for word in lst]
    ge = []
    for b in range(NUM_BINS):
        tot = None
        for w, word in planes:
            term = ((word >> b) & 1) << w
            tot = term if tot is None else tot + term
        ge.append(jnp.sum(tot))
    ge.append(jnp.int32(0))
    counts = [(ge[b] - ge[b + 1]).astype(jnp.float32) for b in range(NUM_BINS)]
    counts += [jnp.float32(0.0)] * (_PAD - NUM_BINS)
    h = jnp.stack(counts).reshape(1, _PAD)

    @pl.when(i == 0)
    def _():
        hist_ref[...] = h

    @pl.when(i > 0)
    def _():
        hist_ref[...] += h


def _loss_body(num_calc, hist_ref, p_ref, t_ref, out_ref):
    counts = hist_ref[...]
    nvalid = jnp.sum((counts > 0).astype(jnp.float32))
    scale = num_calc * nvalid
    recip = jnp.where(counts > 0, scale / jnp.maximum(counts, 1.0), 0.0)

    p = p_ref[...]
    t = t_ref[...]
    bm, bn = p.shape
    x = jnp.abs(p - t) * _SCALE
    idx = x.astype(jnp.int32)
    table = jnp.concatenate(
        [recip, jnp.zeros((1, 128 - _PAD), jnp.float32)], axis=1
    )
    table = jnp.broadcast_to(table, (bm, 128))
    cols = []
    for k in range(bn // 128):
        idxk = idx[:, k * 128 : (k + 1) * 128]
        cols.append(
            jnp.take_along_axis(table, idxk, axis=1, mode="promise_in_bounds")
        )
    w = jnp.concatenate(cols, axis=1)

    pc = jnp.clip(p, EPS_CLIP, 1.0 - EPS_CLIP)
    bce = -(t * jnp.log(pc) + (1.0 - t) * jnp.log(1.0 - pc))
    out_ref[...] = bce * w


def kernel(pconf, gconf):
    m, n = pconf.shape

    bm1 = 256
    blk1 = pl.BlockSpec((bm1, n), lambda i: (i, 0))
    hist = pl.pallas_call(
        _hist_body,
        grid=(m // bm1,),
        in_specs=[blk1, blk1],
        out_specs=pl.BlockSpec((1, _PAD), lambda i: (0, 0)),
        out_shape=jax.ShapeDtypeStruct((1, _PAD), jnp.float32),
    )(pconf, gconf)

    bm2 = 512
    blk2 = pl.BlockSpec((bm2, n), lambda i: (i, 0))
    num_calc = np.float32(m * n)
    loss = pl.pallas_call(
        functools.partial(_loss_body, num_calc),
        grid=(m // bm2,),
        in_specs=[pl.BlockSpec((1, _PAD), lambda i: (0, 0)), blk2, blk2],
        out_specs=blk2,
        out_shape=jax.ShapeDtypeStruct((m, n), jnp.float32),
    )(hist, pconf, gconf)
    return loss


# speedup vs baseline: 5.3145x; 1.0392x over previous
import functools

import jax
import jax.numpy as jnp
import numpy as np
from jax.experimental import pallas as pl

NUM_BINS = 30
EPS_CLIP = 1e-7
_F16_EPS = float(np.finfo(np.float16).eps)
_SCALE = NUM_BINS - _F16_EPS
_PAD = 32


def _csa_push(acc, w, word):
    lst = acc.setdefault(w, [])
    lst.append(word)
    while len(lst) >= 3:
        a = lst.pop()
        b = lst.pop()
        c = lst.pop()
        t = a ^ b
        lst.append(t ^ c)
        _csa_push(acc, w + 1, (a & b) | (t & c))


def _hist_body(p_ref, t_ref, hist_ref):
    i = pl.program_id(0)
    bm, bn = p_ref.shape
    acc = {}
    for r in range(0, bm, 8):
        p = p_ref[pl.ds(r, 8), :]
        t = t_ref[pl.ds(r, 8), :]
        ind = (jnp.abs(p - t) * _SCALE).astype(jnp.int32)
        pw = (jnp.int32(2) << ind) - 1
        for k in range(0, bn, 128):
            _csa_push(acc, 0, pw[:, k : k + 128])

    planes = [(w, word) for w, lst in acc.items() for word in lst]
    ge = []
    for b in range(NUM_BINS):
        tot = None
        for w, word in planes:
            term = ((word >> b) & 1) << w
            tot = term if tot is None else tot + term
        ge.append(jnp.sum(tot))
    ge.append(jnp.int32(0))
    counts = [(ge[b] - ge[b + 1]).astype(jnp.float32) for b in range(NUM_BINS)]
    counts += [jnp.float32(0.0)] * (_PAD - NUM_BINS)
    h = jnp.stack(counts).reshape(1, _PAD)

    @pl.when(i == 0)
    def _():
        hist_ref[...] = h

    @pl.when(i > 0)
    def _():
        hist_ref[...] += h


def _loss_body(num_calc, hist_ref, p_ref, t_ref, out_ref):
    counts = hist_ref[...]
    nvalid = jnp.sum((counts > 0).astype(jnp.float32))
    scale = num_calc * nvalid
    recip = jnp.where(counts > 0, scale / jnp.maximum(counts, 1.0), 0.0)

    p = p_ref[...]
    t = t_ref[...]
    bm, bn = p.shape
    x = jnp.abs(p - t) * _SCALE
    idx = x.astype(jnp.int32)
    table = jnp.concatenate(
        [recip, jnp.zeros((1, 128 - _PAD), jnp.float32)], axis=1
    )
    table = jnp.broadcast_to(table, (bm, 128))
    cols = []
    for k in range(bn // 128):
        idxk = idx[:, k * 128 : (k + 1) * 128]
        cols.append(
            jnp.take_along_axis(table, idxk, axis=1, mode="promise_in_bounds")
        )
    w = jnp.concatenate(cols, axis=1)

    pc = jnp.clip(p, EPS_CLIP, 1.0 - EPS_CLIP)
    bce = -(t * jnp.log(pc) + (1.0 - t) * jnp.log(1.0 - pc))
    out_ref[...] = bce * w


def kernel(pconf, gconf):
    m, n = pconf.shape

    bm1 = 512
    blk1 = pl.BlockSpec((bm1, n), lambda i: (i, 0))
    hist = pl.pallas_call(
        _hist_body,
        grid=(m // bm1,),
        in_specs=[blk1, blk1],
        out_specs=pl.BlockSpec((1, _PAD), lambda i: (0, 0)),
        out_shape=jax.ShapeDtypeStruct((1, _PAD), jnp.float32),
    )(pconf, gconf)

    bm2 = 512
    blk2 = pl.BlockSpec((bm2, n), lambda i: (i, 0))
    num_calc = np.float32(m * n)
    loss = pl.pallas_call(
        functools.partial(_loss_body, num_calc),
        grid=(m // bm2,),
        in_specs=[pl.BlockSpec((1, _PAD), lambda i: (0, 0)), blk2, blk2],
        out_specs=blk2,
        out_shape=jax.ShapeDtypeStruct((m, n), jnp.float32),
    )(hist, pconf, gconf)
    return loss


# speedup vs baseline: 5.3557x; 1.0077x over previous
import functools

import jax
import jax.numpy as jnp
import numpy as np
from jax.experimental import pallas as pl

NUM_BINS = 30
EPS_CLIP = 1e-7
_F16_EPS = float(np.finfo(np.float16).eps)
_SCALE = NUM_BINS - _F16_EPS
_PAD = 32


def _csa_push(acc, w, word):
    lst = acc.setdefault(w, [])
    lst.append(word)
    while len(lst) >= 3:
        a = lst.pop()
        b = lst.pop()
        c = lst.pop()
        t = a ^ b
        lst.append(t ^ c)
        _csa_push(acc, w + 1, (a & b) | (t & c))


def _hist_body(p_ref, t_ref, hist_ref):
    i = pl.program_id(0)
    bm, bn = p_ref.shape
    acc = {}
    for r in range(0, bm, 8):
        p = p_ref[pl.ds(r, 8), :]
        t = t_ref[pl.ds(r, 8), :]
        ind = (jnp.abs(p - t) * _SCALE).astype(jnp.int32)
        pw = jnp.int32(1) << ind
        for k in range(0, bn, 128):
            _csa_push(acc, 0, pw[:, k : k + 128])

    planes = [(w, word) for w, lst in acc.items() for word in lst]
    counts = []
    for b in range(NUM_BINS):
        tot = None
        for w, word in planes:
            term = ((word >> b) & 1) << w
            tot = term if tot is None else tot + term
        counts.append(jnp.sum(tot).astype(jnp.float32))
    counts += [jnp.float32(0.0)] * (_PAD - NUM_BINS)
    h = jnp.stack(counts).reshape(1, _PAD)

    @pl.when(i == 0)
    def _():
        hist_ref[...] = h

    @pl.when(i > 0)
    def _():
        hist_ref[...] += h


def _loss_body(num_calc, hist_ref, p_ref, t_ref, out_ref):
    counts = hist_ref[...]
    nvalid = jnp.sum((counts > 0).astype(jnp.float32))
    scale = num_calc * nvalid
    recip = jnp.where(counts > 0, scale / jnp.maximum(counts, 1.0), 0.0)

    p = p_ref[...]
    t = t_ref[...]
    bm, bn = p.shape
    x = jnp.abs(p - t) * _SCALE
    idx = x.astype(jnp.int32)
    table = jnp.concatenate(
        [recip, jnp.zeros((1, 128 - _PAD), jnp.float32)], axis=1
    )
    table = jnp.broadcast_to(table, (bm, 128))
    cols = []
    for k in range(bn // 128):
        idxk = idx[:, k * 128 : (k + 1) * 128]
        cols.append(
            jnp.take_along_axis(table, idxk, axis=1, mode="promise_in_bounds")
        )
    w = jnp.concatenate(cols, axis=1)

    pc = jnp.clip(p, EPS_CLIP, 1.0 - EPS_CLIP)
    bce = -(t * jnp.log(pc) + (1.0 - t) * jnp.log(1.0 - pc))
    out_ref[...] = bce * w


def kernel(pconf, gconf):
    m, n = pconf.shape

    bm1 = 512
    blk1 = pl.BlockSpec((bm1, n), lambda i: (i, 0))
    hist = pl.pallas_call(
        _hist_body,
        grid=(m // bm1,),
        in_specs=[blk1, blk1],
        out_specs=pl.BlockSpec((1, _PAD), lambda i: (0, 0)),
        out_shape=jax.ShapeDtypeStruct((1, _PAD), jnp.float32),
    )(pconf, gconf)

    bm2 = 512
    blk2 = pl.BlockSpec((bm2, n), lambda i: (i, 0))
    num_calc = np.float32(m * n)
    loss = pl.pallas_call(
        functools.partial(_loss_body, num_calc),
        grid=(m // bm2,),
        in_specs=[pl.BlockSpec((1, _PAD), lambda i: (0, 0)), blk2, blk2],
        out_specs=blk2,
        out_shape=jax.ShapeDtypeStruct((m, n), jnp.float32),
    )(hist, pconf, gconf)
    return loss


# speedup vs baseline: 5.3592x; 1.0006x over previous
import functools

import jax
import jax.numpy as jnp
import numpy as np
from jax.experimental import pallas as pl
from jax.experimental.pallas import tpu as pltpu

NUM_BINS = 30
EPS_CLIP = 1e-7
_F16_EPS = float(np.finfo(np.float16).eps)
_SCALE = NUM_BINS - _F16_EPS
_PAD = 32


def _csa_push(acc, w, word):
    lst = acc.setdefault(w, [])
    lst.append(word)
    while len(lst) >= 3:
        a = lst.pop()
        b = lst.pop()
        c = lst.pop()
        t = a ^ b
        lst.append(t ^ c)
        _csa_push(acc, w + 1, (a & b) | (t & c))


def _hist_body(p_ref, t_ref, hist_ref):
    i = pl.program_id(0)
    bm, bn = p_ref.shape
    acc = {}
    for r in range(0, bm, 8):
        p = p_ref[pl.ds(r, 8), :]
        t = t_ref[pl.ds(r, 8), :]
        ind = (jnp.abs(p - t) * _SCALE).astype(jnp.int32)
        pw = jnp.int32(1) << ind
        for k in range(0, bn, 128):
            _csa_push(acc, 0, pw[:, k : k + 128])

    planes = [(w, word) for w, lst in acc.items() for word in lst]
    counts = []
    for b in range(NUM_BINS):
        tot = None
        for w, word in planes:
            term = ((word >> b) & 1) << w
            tot = term if tot is None else tot + term
        counts.append(jnp.sum(tot).astype(jnp.float32))
    counts += [jnp.float32(0.0)] * (_PAD - NUM_BINS)
    h = jnp.stack(counts).reshape(1, _PAD)

    @pl.when(i == 0)
    def _():
        hist_ref[...] = h

    @pl.when(i > 0)
    def _():
        hist_ref[...] += h


def _loss_body(num_calc, hist_ref, p_ref, t_ref, out_ref):
    counts = hist_ref[...]
    nvalid = jnp.sum((counts > 0).astype(jnp.float32))
    scale = num_calc * nvalid
    recip = jnp.where(counts > 0, scale / jnp.maximum(counts, 1.0), 0.0)

    p = p_ref[...]
    t = t_ref[...]
    bm, bn = p.shape
    x = jnp.abs(p - t) * _SCALE
    idx = x.astype(jnp.int32)
    table = jnp.concatenate(
        [recip, jnp.zeros((1, 128 - _PAD), jnp.float32)], axis=1
    )
    table = jnp.broadcast_to(table, (bm, 128))
    cols = []
    for k in range(bn // 128):
        idxk = idx[:, k * 128 : (k + 1) * 128]
        cols.append(
            jnp.take_along_axis(table, idxk, axis=1, mode="promise_in_bounds")
        )
    w = jnp.concatenate(cols, axis=1)

    pc = jnp.clip(p, EPS_CLIP, 1.0 - EPS_CLIP)
    bce = -(t * jnp.log(pc) + (1.0 - t) * jnp.log(1.0 - pc))
    out_ref[...] = bce * w


def kernel(pconf, gconf):
    m, n = pconf.shape

    bm1 = 512
    blk1 = pl.BlockSpec((bm1, n), lambda i: (i, 0))
    hist = pl.pallas_call(
        _hist_body,
        grid=(m // bm1,),
        in_specs=[blk1, blk1],
        out_specs=pl.BlockSpec((1, _PAD), lambda i: (0, 0)),
        out_shape=jax.ShapeDtypeStruct((1, _PAD), jnp.float32),
    )(pconf, gconf)

    bm2 = 512
    blk2 = pl.BlockSpec((bm2, n), lambda i: (i, 0))
    num_calc = np.float32(m * n)
    loss = pl.pallas_call(
        functools.partial(_loss_body, num_calc),
        grid=(m // bm2,),
        in_specs=[pl.BlockSpec((1, _PAD), lambda i: (0, 0)), blk2, blk2],
        out_specs=blk2,
        out_shape=jax.ShapeDtypeStruct((m, n), jnp.float32),
    )(hist, pconf, gconf)
    return loss


# speedup vs baseline: 5.3594x; 1.0000x over previous
import functools

import jax
import jax.numpy as jnp
import numpy as np
from jax.experimental import pallas as pl

NUM_BINS = 30
EPS_CLIP = 1e-7
_F16_EPS = float(np.finfo(np.float16).eps)
_SCALE = NUM_BINS - _F16_EPS
_PAD = 32


def _csa_push(acc, w, word):
    lst = acc.setdefault(w, [])
    lst.append(word)
    while len(lst) >= 3:
        a = lst.pop()
        b = lst.pop()
        c = lst.pop()
        t = a ^ b
        lst.append(t ^ c)
        _csa_push(acc, w + 1, (a & b) | (t & c))


def _hist_body(p_ref, t_ref, hist_ref):
    i = pl.program_id(0)
    bm, bn = p_ref.shape
    acc = {}
    for r in range(0, bm, 8):
        p = p_ref[pl.ds(r, 8), :]
        t = t_ref[pl.ds(r, 8), :]
        ind = (jnp.abs(p - t) * _SCALE).astype(jnp.int32)
        pw = jnp.int32(1) << ind
        for k in range(0, bn, 128):
            _csa_push(acc, 0, pw[:, k : k + 128])

    planes = [(w, word) for w, lst in acc.items() for word in lst]
    counts = []
    for b in range(NUM_BINS):
        tot = None
        for w, word in planes:
            term = ((word >> b) & 1) << w
            tot = term if tot is None else tot + term
        counts.append(jnp.sum(tot).astype(jnp.float32))
    counts += [jnp.float32(0.0)] * (_PAD - NUM_BINS)
    h = jnp.stack(counts).reshape(1, _PAD)

    @pl.when(i == 0)
    def _():
        hist_ref[...] = h

    @pl.when(i > 0)
    def _():
        hist_ref[...] += h


def _loss_body(num_calc, hist_ref, p_ref, t_ref, out_ref):
    counts = hist_ref[...]
    nvalid = jnp.sum((counts > 0).astype(jnp.float32))
    scale = num_calc * nvalid
    recip = jnp.where(counts > 0, scale / jnp.maximum(counts, 1.0), 0.0)

    p = p_ref[...]
    t = t_ref[...]
    bm, bn = p.shape
    x = jnp.abs(p - t) * _SCALE
    idx = x.astype(jnp.int32)
    table = jnp.concatenate(
        [recip, jnp.zeros((1, 128 - _PAD), jnp.float32)], axis=1
    )
    table = jnp.broadcast_to(table, (bm, 128))
    cols = []
    for k in range(bn // 128):
        idxk = idx[:, k * 128 : (k + 1) * 128]
        cols.append(
            jnp.take_along_axis(table, idxk, axis=1, mode="promise_in_bounds")
        )
    w = jnp.concatenate(cols, axis=1)

    pc = jnp.clip(p, EPS_CLIP, 1.0 - EPS_CLIP)
    bce = -(t * jnp.log(pc) + (1.0 - t) * jnp.log(1.0 - pc))
    out_ref[...] = bce * w


def kernel(pconf, gconf):
    m, n = pconf.shape

    bm1 = 512
    blk1 = pl.BlockSpec((bm1, n), lambda i: (i, 0))
    hist = pl.pallas_call(
        _hist_body,
        grid=(m // bm1,),
        in_specs=[blk1, blk1],
        out_specs=pl.BlockSpec((1, _PAD), lambda i: (0, 0)),
        out_shape=jax.ShapeDtypeStruct((1, _PAD), jnp.float32),
    )(pconf, gconf)

    bm2 = 512
    blk2 = pl.BlockSpec((bm2, n), lambda i: (i, 0))
    num_calc = np.float32(m * n)
    loss = pl.pallas_call(
        functools.partial(_loss_body, num_calc),
        grid=(m // bm2,),
        in_specs=[pl.BlockSpec((1, _PAD), lambda i: (0, 0)), blk2, blk2],
        out_specs=blk2,
        out_shape=jax.ShapeDtypeStruct((m, n), jnp.float32),
    )(hist, pconf, gconf)
    return loss
